# Initial kernel scaffold; baseline (speedup 1.0000x reference)
import jax
import jax.numpy as jnp
from jax.experimental import pallas as pl


def kernel(rm, psm, pos_equal_one, neg_equal_one, targets):
    raise NotImplementedError("write your pallas kernel here")


# speedup vs baseline: 25.7801x; 25.7801x over previous
import jax
import jax.numpy as jnp
from jax.experimental import pallas as pl

_NEG_RATIO = 3
_ALPHA = 1.5
_BETA = 1.0
_GAMMA = 2.0
_HI_BITS = 0x41800000


def _reg_kernel(rm_ref, tg_ref, pos_ref, e_ref, num_ref, psum_ref):
    rm = rm_ref[...]
    tg = tg_ref[...]
    pos = pos_ref[...]
    th, w = pos.shape[1], pos.shape[2]
    mask = jnp.dot(pos.reshape(th * w, 10), e_ref[...],
                   preferred_element_type=jnp.float32)
    d = (rm - tg).reshape(th * w, 70) * mask
    ad = jnp.abs(d)
    sl1 = jnp.where(ad < 1.0, 0.5 * d * d, ad - 0.5)
    num_ref[...] = jnp.broadcast_to(jnp.sum(sl1), (1, 1, 1, 1))
    psum_ref[...] = jnp.broadcast_to(jnp.sum(pos), (1, 1, 1, 1))


def _cls_kernel(psm_ref, pos_ref, neg_ref, clsp_ref, topk_ref, k_ref):
    x = psm_ref[...]
    pos = pos_ref[...]
    neg = neg_ref[...]
    p = jax.nn.sigmoid(x)
    clsp = jnp.sum(-pos * jnp.log(p + 1e-6))
    fpos = jnp.sum(pos)
    n = jnp.int32(x.size)
    k = jnp.minimum((_NEG_RATIO * (fpos + 1.0)).astype(jnp.int32), n)
    fl = jnp.maximum(-neg * jnp.log(1.0 - p + 1e-6), 0.0)
    bits = jax.lax.bitcast_convert_type(fl, jnp.int32)

    def body(_, carry):
        lo, hi = carry
        mid = lo + (hi - lo) // 2
        c = jnp.sum((bits > mid).astype(jnp.int32))
        shrink = c < k
        return (jnp.where(shrink, lo, mid + 1),
                jnp.where(shrink, mid, hi))

    lo, _ = jax.lax.fori_loop(
        0, 31, body, (jnp.int32(0), jnp.int32(_HI_BITS)))
    t = jax.lax.bitcast_convert_type(lo, jnp.float32)
    gt = bits > lo
    cnt = jnp.sum(gt.astype(jnp.int32))
    topk = jnp.sum(jnp.where(gt, fl, 0.0)) + (k - cnt).astype(jnp.float32) * t
    clsp_ref[...] = jnp.broadcast_to(clsp, (1, 1, 1))
    topk_ref[...] = jnp.broadcast_to(topk, (1, 1, 1))
    k_ref[...] = jnp.broadcast_to(k.astype(jnp.float32), (1, 1, 1))


def kernel(rm, psm, pos_equal_one, neg_equal_one, targets):
    b, a, h, w = psm.shape
    nframe = a // 2
    rm_t = jnp.transpose(rm, (0, 2, 3, 1))
    pos_cf = jnp.transpose(pos_equal_one, (0, 3, 1, 2))
    neg_cf = jnp.transpose(neg_equal_one, (0, 3, 1, 2))
    e = (jnp.arange(7 * a, dtype=jnp.int32)[None, :] // 7
         == jnp.arange(a, dtype=jnp.int32)[:, None]).astype(jnp.float32)

    nh = 8
    th = h // nh
    num, psum = pl.pallas_call(
        _reg_kernel,
        grid=(b, nh),
        in_specs=[
            pl.BlockSpec((1, th, w, 7 * a), lambda i, j: (i, j, 0, 0)),
            pl.BlockSpec((1, th, w, 7 * a), lambda i, j: (i, j, 0, 0)),
            pl.BlockSpec((1, th, w, a), lambda i, j: (i, j, 0, 0)),
            pl.BlockSpec((a, 7 * a), lambda i, j: (0, 0)),
        ],
        out_specs=[
            pl.BlockSpec((1, 1, 1, 1), lambda i, j: (i, j, 0, 0)),
            pl.BlockSpec((1, 1, 1, 1), lambda i, j: (i, j, 0, 0)),
        ],
        out_shape=[
            jax.ShapeDtypeStruct((b, nh, 1, 1), jnp.float32),
            jax.ShapeDtypeStruct((b, nh, 1, 1), jnp.float32),
        ],
    )(rm_t, targets, pos_equal_one, e)

    nf = b * nframe
    clsp, topk, kf = pl.pallas_call(
        _cls_kernel,
        grid=(nf,),
        in_specs=[
            pl.BlockSpec((1, 2, h, w), lambda f: (f // nframe, f % nframe, 0, 0)),
            pl.BlockSpec((1, 2, h, w), lambda f: (f // nframe, f % nframe, 0, 0)),
            pl.BlockSpec((1, 2, h, w), lambda f: (f // nframe, f % nframe, 0, 0)),
        ],
        out_specs=[
            pl.BlockSpec((1, 1, 1), lambda f: (f, 0, 0)),
            pl.BlockSpec((1, 1, 1), lambda f: (f, 0, 0)),
            pl.BlockSpec((1, 1, 1), lambda f: (f, 0, 0)),
        ],
        out_shape=[
            jax.ShapeDtypeStruct((nf, 1, 1), jnp.float32),
            jax.ShapeDtypeStruct((nf, 1, 1), jnp.float32),
            jax.ShapeDtypeStruct((nf, 1, 1), jnp.float32),
        ],
    )(psm, pos_cf, neg_cf)

    pos_sum = jnp.sum(psum)
    reg_loss = _GAMMA * jnp.sum(num) / (pos_sum + 1e-6)
    cls_pos_loss = _ALPHA * jnp.sum(clsp) / (pos_sum + 1e-6)
    cls_neg_loss = _BETA * jnp.sum(topk) / (jnp.sum(kf) + 1e-6)
    conf_loss = cls_pos_loss + cls_neg_loss
    return (conf_loss, reg_loss, cls_pos_loss, cls_neg_loss)
